# baseline (speedup 1.0000x reference)
import functools

import jax
import jax.numpy as jnp
from jax import lax
from jax.experimental import pallas as pl
from jax.experimental.pallas import tpu as pltpu
from jax.experimental.pallas import tpu_sc as plsc

_NC = 2
_NS = 16
_N = 10000
_E = 320000
_D = 128
_NP = 10240

_EC = _E // _NS
_CH = 80
_NCHUNK = _EC // _CH

_ROWS_PT = _N // _NS
_PADROWS_PT = _NP // _NS

@functools.cache
def _sc_mesh():
    return plsc.VectorSubcoreMesh(
        core_axis_name="c", subcore_axis_name="s",
        num_cores=_NC, num_subcores=_NS,
    )


def _deg_body(dst1, ew1, dst2, ew2, deg1, deg2, acc, idx_v, val_v):
    c = lax.axis_index("c")
    s = lax.axis_index("s")

    def _fill(i, _):
        val_v[pl.ds(i * 16, 16)] = jnp.full((16,), 1.0, jnp.float32)
        return 0

    lax.fori_loop(0, _CH // 16, _fill, 0)

    def _init(i, _):
        pltpu.sync_copy(val_v, acc.at[pl.ds(s * _PADROWS_PT + i * _CH, _CH)])
        return 0

    lax.fori_loop(0, _PADROWS_PT // _CH, _init, 0)
    plsc.subcore_barrier()

    def _run(dst_hbm, ew_hbm):
        def _step(k, _):
            b = s * _EC + k * _CH
            pltpu.sync_copy(dst_hbm.at[pl.ds(b, _CH)], idx_v)
            pltpu.sync_copy(ew_hbm.at[pl.ds(b, _CH)], val_v)
            pltpu.sync_copy(val_v, acc.at[idx_v], add=True)
            return 0

        lax.fori_loop(0, _NCHUNK, _step, 0)

    @pl.when(c == 0)
    def _():
        _run(dst1, ew1)

    @pl.when(c == 1)
    def _():
        _run(dst2, ew2)

    plsc.subcore_barrier()

    @pl.when(c == 0)
    def _():
        pltpu.sync_copy(acc.at[pl.ds(s * _PADROWS_PT, _PADROWS_PT)],
                        deg1.at[pl.ds(s * _PADROWS_PT, _PADROWS_PT)])

    @pl.when(c == 1)
    def _():
        pltpu.sync_copy(acc.at[pl.ds(s * _PADROWS_PT, _PADROWS_PT)],
                        deg2.at[pl.ds(s * _PADROWS_PT, _PADROWS_PT)])


@functools.cache
def _deg_call():
    return pl.kernel(
        _deg_body,
        out_type=[jax.ShapeDtypeStruct((_NP,), jnp.float32),
                  jax.ShapeDtypeStruct((_NP,), jnp.float32)],
        mesh=_sc_mesh(),
        scratch_types=[
            pltpu.VMEM_SHARED((_NP,), jnp.float32),
            pltpu.VMEM((_CH,), jnp.int32),
            pltpu.VMEM((_CH,), jnp.float32),
        ],
    )


_ACH = 128
_EPAD = 327680
_ER = _EPAD // _ACH
_TCH = _ER // _NS


_NSUB = 4
_SUBR = _ACH // _NSUB


def _agg_body(u1, src1, dst1, ew1, u2, src2, dst2, ew2, v1, v2,
              acc, srci, dsti, ewv, rows,
              gs0, gs1, gs2, gs3, gs4, gs5, gs6, gs7, isem0, isem1):
    gsems = ((gs0, gs1, gs2, gs3), (gs4, gs5, gs6, gs7))
    c = lax.axis_index("c")
    s = lax.axis_index("s")

    def _zrow(r, _):
        for j in range(_D // 16):
            rows[0, r, pl.ds(j * 16, 16)] = jnp.zeros((16,), jnp.float32)
        return 0

    lax.fori_loop(0, _ACH, _zrow, 0)

    def _zinit(i, _):
        pltpu.sync_copy(rows.at[0],
                        acc.at[pl.ds(s * _PADROWS_PT + i * _ACH, _ACH)])
        return 0

    lax.fori_loop(0, _PADROWS_PT // _ACH, _zinit, 0)
    plsc.subcore_barrier()

    def _run(u_hbm, src_hbm, dst_hbm, ew_hbm):
        ebase = s * _TCH * _ACH

        def _ld(k, slot, sem):
            off = ebase + k * _ACH
            pltpu.async_copy(src_hbm.at[pl.ds(off, _ACH)], srci.at[slot], sem)
            pltpu.async_copy(dst_hbm.at[pl.ds(off, _ACH)], dsti.at[slot], sem)
            pltpu.async_copy(ew_hbm.at[pl.ds(off, _ACH)], ewv.at[slot], sem)

        def _ldwait(k, slot, sem):
            off = ebase + k * _ACH
            pltpu.make_async_copy(
                src_hbm.at[pl.ds(off, _ACH)], srci.at[slot], sem).wait()
            pltpu.make_async_copy(
                dst_hbm.at[pl.ds(off, _ACH)], dsti.at[slot], sem).wait()
            pltpu.make_async_copy(
                ew_hbm.at[pl.ds(off, _ACH)], ewv.at[slot], sem).wait()

        def _gissue(slot):
            for q in range(_NSUB):
                sl = pl.ds(q * _SUBR, _SUBR)
                pltpu.async_copy(u_hbm.at[srci.at[slot, sl]],
                                 rows.at[slot, sl], gsems[slot][q])

        def _gwait(slot):
            for q in range(_NSUB):
                sl = pl.ds(q * _SUBR, _SUBR)
                pltpu.make_async_copy(u_hbm.at[srci.at[slot, sl]],
                                      rows.at[slot, sl],
                                      gsems[slot][q]).wait()

        _ld(0, 0, isem0)
        _ldwait(0, 0, isem0)
        _gissue(0)
        _ld(1, 1, isem1)

        def _pair(kk, _):
            for b in range(2):
                k = kk * 2 + b
                i_this, i_next = (isem0, isem1) if b == 0 else (isem1, isem0)
                _gwait(b)

                @pl.when(k < _TCH - 1)
                def _():
                    _ldwait(k + 1, 1 - b, i_next)
                    _gissue(1 - b)

                def _grp(g, _):
                    wv = ewv[b, pl.ds(g * 16, 16)]
                    e0 = g * 16
                    for l in range(16):
                        w = jnp.take_along_axis(
                            wv, jnp.full((16,), l, jnp.int32), axis=0)
                        for j in range(_D // 16):
                            sl = pl.ds(j * 16, 16)
                            rows[b, e0 + l, sl] = rows[b, e0 + l, sl] * w
                    return 0

                lax.fori_loop(0, _ACH // 16, _grp, 0)
                pltpu.sync_copy(rows.at[b], acc.at[dsti.at[b]], add=True)

                @pl.when(k < _TCH - 2)
                def _():
                    _ld(k + 2, b, i_this)
            return 0

        lax.fori_loop(0, _TCH // 2, _pair, 0)

    @pl.when(c == 0)
    def _():
        _run(u1, src1, dst1, ew1)

    @pl.when(c == 1)
    def _():
        _run(u2, src2, dst2, ew2)

    plsc.subcore_barrier()

    @pl.when(c == 0)
    def _():
        pltpu.sync_copy(acc.at[pl.ds(s * _PADROWS_PT, _PADROWS_PT)],
                        v1.at[pl.ds(s * _PADROWS_PT, _PADROWS_PT)])

    @pl.when(c == 1)
    def _():
        pltpu.sync_copy(acc.at[pl.ds(s * _PADROWS_PT, _PADROWS_PT)],
                        v2.at[pl.ds(s * _PADROWS_PT, _PADROWS_PT)])


@functools.cache
def _agg_call():
    return pl.kernel(
        _agg_body,
        out_type=[jax.ShapeDtypeStruct((_NP, _D), jnp.float32),
                  jax.ShapeDtypeStruct((_NP, _D), jnp.float32)],
        mesh=_sc_mesh(),
        scratch_types=[
            pltpu.VMEM_SHARED((_NP, _D), jnp.float32),
            pltpu.VMEM((2, _ACH), jnp.int32),
            pltpu.VMEM((2, _ACH), jnp.int32),
            pltpu.VMEM((2, _ACH), jnp.float32),
            pltpu.VMEM((2, _ACH, _D), jnp.float32),
        ] + [pltpu.SemaphoreType.DMA] * 10,
    )


def _pad1d(x):
    pad = jnp.zeros((_EPAD - _E,), x.dtype)
    return jnp.concatenate([x, pad])


_BLK = 1000


def _lin_body(deg_ref, fm_ref, w_ref, u_ref, dinv_ref):
    dinv = lax.rsqrt(deg_ref[...])
    u_ref[...] = dinv * jnp.dot(fm_ref[...], w_ref[...],
                                preferred_element_type=jnp.float32)
    dinv_ref[...] = dinv


def _lin(deg2d, fm, W):
    return pl.pallas_call(
        _lin_body,
        grid=(_N // _BLK,),
        in_specs=[
            pl.BlockSpec((_BLK, 1), lambda i: (i, 0)),
            pl.BlockSpec((_BLK, _D), lambda i: (i, 0)),
            pl.BlockSpec((_D, _D), lambda i: (0, 0)),
        ],
        out_specs=[
            pl.BlockSpec((_BLK, _D), lambda i: (i, 0)),
            pl.BlockSpec((_BLK, 1), lambda i: (i, 0)),
        ],
        out_shape=[jax.ShapeDtypeStruct((_N, _D), jnp.float32),
                   jax.ShapeDtypeStruct((_N, 1), jnp.float32)],
    )(deg2d, fm, W)


def _layer_body(dinv_ref, v_ref, u_ref, b_ref, w2_ref, x_ref, u2_ref):
    dinv = dinv_ref[...]
    x = dinv * (v_ref[...] + u_ref[...]) + b_ref[...]
    x_ref[...] = x
    u2_ref[...] = dinv * jnp.dot(x, w2_ref[...],
                                 preferred_element_type=jnp.float32)


def _layer(dinv, v, u, b, W2):
    return pl.pallas_call(
        _layer_body,
        grid=(_N // _BLK,),
        in_specs=[
            pl.BlockSpec((_BLK, 1), lambda i: (i, 0)),
            pl.BlockSpec((_BLK, _D), lambda i: (i, 0)),
            pl.BlockSpec((_BLK, _D), lambda i: (i, 0)),
            pl.BlockSpec((1, _D), lambda i: (0, 0)),
            pl.BlockSpec((_D, _D), lambda i: (0, 0)),
        ],
        out_specs=[
            pl.BlockSpec((_BLK, _D), lambda i: (i, 0)),
            pl.BlockSpec((_BLK, _D), lambda i: (i, 0)),
        ],
        out_shape=[jax.ShapeDtypeStruct((_N, _D), jnp.float32),
                   jax.ShapeDtypeStruct((_N, _D), jnp.float32)],
    )(dinv, v, u, b, W2)


_G = 64


def _final_body(x11_ref, x12_ref, v21_ref, u21_ref, dinv1_ref,
                v22_ref, u22_ref, dinv2_ref, b2_ref, batch_ref,
                m1_ref, bm1_ref, m2_ref, bm2_ref, m3_ref, bm3_ref,
                m4_ref, bm4_ref, out_ref, sums, cnt):
    i = pl.program_id(0)

    @pl.when(i == 0)
    def _():
        sums[...] = jnp.zeros_like(sums)
        cnt[...] = jnp.zeros_like(cnt)

    x21 = dinv1_ref[...] * (v21_ref[...] + u21_ref[...]) + b2_ref[...]
    x22 = dinv2_ref[...] * (v22_ref[...] + u22_ref[...]) + b2_ref[...]
    x = (x12_ref[...] - x11_ref[...]) * (x22 - x21)

    gids = lax.broadcasted_iota(jnp.int32, (1, _G), 1)
    oh = (batch_ref[...] == gids).astype(jnp.float32)
    dn = (((0,), (0,)), ((), ()))
    sums[...] += lax.dot_general(oh, x, dn, preferred_element_type=jnp.float32)
    cnt[...] += lax.dot_general(oh, jnp.ones_like(x), dn,
                                preferred_element_type=jnp.float32)

    @pl.when(i == (_N // _BLK) - 1)
    def _():
        pooled = sums[...] / jnp.maximum(cnt[...], 1.0)
        h = jnp.dot(pooled, m1_ref[...], preferred_element_type=jnp.float32) + bm1_ref[...]
        h = jnp.dot(h, m2_ref[...], preferred_element_type=jnp.float32) + bm2_ref[...]
        h = jnp.dot(h, m3_ref[...], preferred_element_type=jnp.float32) + bm3_ref[...]
        h = jnp.dot(h, m4_ref[...], preferred_element_type=jnp.float32) + bm4_ref[...]
        out_ref[...] = h


def _final(x11, x12, v21, u21, dinv1, v22, u22, dinv2, b2, batch2d,
           M1, bm1, M2, bm2, M3, bm3, M4, bm4):
    row = lambda i: (i, 0)
    full = lambda i: (0, 0)
    return pl.pallas_call(
        _final_body,
        grid=(_N // _BLK,),
        in_specs=[
            pl.BlockSpec((_BLK, _D), row),
            pl.BlockSpec((_BLK, _D), row),
            pl.BlockSpec((_BLK, _D), row),
            pl.BlockSpec((_BLK, _D), row),
            pl.BlockSpec((_BLK, 1), row),
            pl.BlockSpec((_BLK, _D), row),
            pl.BlockSpec((_BLK, _D), row),
            pl.BlockSpec((_BLK, 1), row),
            pl.BlockSpec((1, _D), full),
            pl.BlockSpec((_BLK, 1), row),
            pl.BlockSpec((_D, _D), full),
            pl.BlockSpec((1, _D), full),
            pl.BlockSpec((_D, _D // 2), full),
            pl.BlockSpec((1, _D // 2), full),
            pl.BlockSpec((_D // 2, _D // 4), full),
            pl.BlockSpec((1, _D // 4), full),
            pl.BlockSpec((_D // 4, 1), full),
            pl.BlockSpec((1, 1), full),
        ],
        out_specs=pl.BlockSpec((_G, 1), full),
        out_shape=jax.ShapeDtypeStruct((_G, 1), jnp.float32),
        scratch_shapes=[
            pltpu.VMEM((_G, _D), jnp.float32),
            pltpu.VMEM((_G, _D), jnp.float32),
        ],
    )(x11, x12, v21, u21, dinv1, v22, u22, dinv2, b2, batch2d,
      M1, bm1, M2, bm2, M3, bm3, M4, bm4)


def kernel(edge_index1, edge_weight1, edge_index2, edge_weight2, fm0, fm1,
           batch_tensor, W1, b1, W2, b2, M1, bm1, M2, bm2, M3, bm3, M4, bm4):
    src1, dst1 = edge_index1[0], edge_index1[1]
    src2, dst2 = edge_index2[0], edge_index2[1]

    deg1p, deg2p = _deg_call()(dst1, edge_weight1, dst2, edge_weight2)
    deg1, deg2 = deg1p[:_N], deg2p[:_N]

    sp1, dp1, wp1 = _pad1d(src1), _pad1d(dst1), _pad1d(edge_weight1)
    sp2, dp2, wp2 = _pad1d(src2), _pad1d(dst2), _pad1d(edge_weight2)

    u11, dinv1 = _lin(deg1.reshape(_N, 1), fm0, W1)
    u12, dinv2 = _lin(deg2.reshape(_N, 1), fm1, W1)

    v11p, v12p = _agg_call()(u11, sp1, dp1, wp1, u12, sp2, dp2, wp2)
    v11, v12 = v11p[:_N], v12p[:_N]

    b1r = b1.reshape(1, _D)
    x11, u21 = _layer(dinv1, v11, u11, b1r, W2)
    x12, u22 = _layer(dinv2, v12, u12, b1r, W2)

    v21p, v22p = _agg_call()(u21, sp1, dp1, wp1, u22, sp2, dp2, wp2)
    v21, v22 = v21p[:_N], v22p[:_N]

    return _final(x11, x12, v21, u21, dinv1, v22, u22, dinv2,
                  b2.reshape(1, _D), batch_tensor.reshape(_N, 1),
                  M1, bm1.reshape(1, _D), M2, bm2.reshape(1, _D // 2),
                  M3, bm3.reshape(1, _D // 4), M4, bm4.reshape(1, 1))


# speedup vs baseline: 1.1271x; 1.1271x over previous
import functools

import jax
import jax.numpy as jnp
from jax import lax
from jax.experimental import pallas as pl
from jax.experimental.pallas import tpu as pltpu
from jax.experimental.pallas import tpu_sc as plsc

_NC = 2
_NS = 16
_N = 10000
_E = 320000
_D = 128
_NP = 10240

_ROWS_PT = _N // _NS
_PADROWS_PT = _NP // _NS

@functools.cache
def _sc_mesh():
    return plsc.VectorSubcoreMesh(
        core_axis_name="c", subcore_axis_name="s",
        num_cores=_NC, num_subcores=_NS,
    )


_DCH = 128
_DTC = 327680 // (16 * _DCH)


def _deg_body(dst1, ew1, dst2, ew2, deg1, deg2, acc, idxr, valr,
              isem0, isem1):
    c = lax.axis_index("c")
    s = lax.axis_index("s")

    def _fill(i, _):
        valr[0, pl.ds(i * 16, 16)] = jnp.full((16,), 1.0, jnp.float32)
        return 0

    lax.fori_loop(0, _DCH // 16, _fill, 0)

    def _init(i, _):
        pltpu.sync_copy(valr.at[0],
                        acc.at[pl.ds(s * _PADROWS_PT + i * _DCH, _DCH)])
        return 0

    lax.fori_loop(0, _PADROWS_PT // _DCH, _init, 0)
    plsc.subcore_barrier()

    def _run(dst_hbm, ew_hbm):
        ebase = s * _DTC * _DCH

        def _ld(k, slot, sem):
            off = ebase + k * _DCH
            pltpu.async_copy(dst_hbm.at[pl.ds(off, _DCH)], idxr.at[slot], sem)
            pltpu.async_copy(ew_hbm.at[pl.ds(off, _DCH)], valr.at[slot], sem)

        def _ldwait(k, slot, sem):
            off = ebase + k * _DCH
            pltpu.make_async_copy(
                dst_hbm.at[pl.ds(off, _DCH)], idxr.at[slot], sem).wait()
            pltpu.make_async_copy(
                ew_hbm.at[pl.ds(off, _DCH)], valr.at[slot], sem).wait()

        _ld(0, 0, isem0)
        _ld(1, 1, isem1)

        def _pair(kk, _):
            for b in range(2):
                k = kk * 2 + b
                i_this = isem0 if b == 0 else isem1
                _ldwait(k, b, i_this)
                pltpu.sync_copy(valr.at[b], acc.at[idxr.at[b]], add=True)

                @pl.when(k < _DTC - 2)
                def _():
                    _ld(k + 2, b, i_this)
            return 0

        lax.fori_loop(0, _DTC // 2, _pair, 0)

    @pl.when(c == 0)
    def _():
        _run(dst1, ew1)

    @pl.when(c == 1)
    def _():
        _run(dst2, ew2)

    plsc.subcore_barrier()

    @pl.when(c == 0)
    def _():
        pltpu.sync_copy(acc.at[pl.ds(s * _PADROWS_PT, _PADROWS_PT)],
                        deg1.at[pl.ds(s * _PADROWS_PT, _PADROWS_PT)])

    @pl.when(c == 1)
    def _():
        pltpu.sync_copy(acc.at[pl.ds(s * _PADROWS_PT, _PADROWS_PT)],
                        deg2.at[pl.ds(s * _PADROWS_PT, _PADROWS_PT)])


@functools.cache
def _deg_call():
    return pl.kernel(
        _deg_body,
        out_type=[jax.ShapeDtypeStruct((_NP,), jnp.float32),
                  jax.ShapeDtypeStruct((_NP,), jnp.float32)],
        mesh=_sc_mesh(),
        scratch_types=[
            pltpu.VMEM_SHARED((_NP,), jnp.float32),
            pltpu.VMEM((2, _DCH), jnp.int32),
            pltpu.VMEM((2, _DCH), jnp.float32),
            pltpu.SemaphoreType.DMA,
            pltpu.SemaphoreType.DMA,
        ],
    )


_ACH = 128
_EPAD = 327680
_ER = _EPAD // _ACH
_TCH = _ER // _NS


_NSUB = 4
_SUBR = _ACH // _NSUB


def _agg_body(u1, src1, dst1, ew1, u2, src2, dst2, ew2, v1, v2,
              acc, srci, dsti, ewv, rows,
              gs0, gs1, gs2, gs3, gs4, gs5, gs6, gs7, isem0, isem1):
    gsems = ((gs0, gs1, gs2, gs3), (gs4, gs5, gs6, gs7))
    c = lax.axis_index("c")
    s = lax.axis_index("s")

    def _zrow(r, _):
        for j in range(_D // 16):
            rows[0, r, pl.ds(j * 16, 16)] = jnp.zeros((16,), jnp.float32)
        return 0

    lax.fori_loop(0, _ACH, _zrow, 0)

    def _zinit(i, _):
        pltpu.sync_copy(rows.at[0],
                        acc.at[pl.ds(s * _PADROWS_PT + i * _ACH, _ACH)])
        return 0

    lax.fori_loop(0, _PADROWS_PT // _ACH, _zinit, 0)
    plsc.subcore_barrier()

    def _run(u_hbm, src_hbm, dst_hbm, ew_hbm):
        ebase = s * _TCH * _ACH

        def _ld(k, slot, sem):
            off = ebase + k * _ACH
            pltpu.async_copy(src_hbm.at[pl.ds(off, _ACH)], srci.at[slot], sem)
            pltpu.async_copy(dst_hbm.at[pl.ds(off, _ACH)], dsti.at[slot], sem)
            pltpu.async_copy(ew_hbm.at[pl.ds(off, _ACH)], ewv.at[slot], sem)

        def _ldwait(k, slot, sem):
            off = ebase + k * _ACH
            pltpu.make_async_copy(
                src_hbm.at[pl.ds(off, _ACH)], srci.at[slot], sem).wait()
            pltpu.make_async_copy(
                dst_hbm.at[pl.ds(off, _ACH)], dsti.at[slot], sem).wait()
            pltpu.make_async_copy(
                ew_hbm.at[pl.ds(off, _ACH)], ewv.at[slot], sem).wait()

        def _gissue(slot):
            for q in range(_NSUB):
                sl = pl.ds(q * _SUBR, _SUBR)
                pltpu.async_copy(u_hbm.at[srci.at[slot, sl]],
                                 rows.at[slot, sl], gsems[slot][q])

        def _gwait(slot):
            for q in range(_NSUB):
                sl = pl.ds(q * _SUBR, _SUBR)
                pltpu.make_async_copy(u_hbm.at[srci.at[slot, sl]],
                                      rows.at[slot, sl],
                                      gsems[slot][q]).wait()

        _ld(0, 0, isem0)
        _ldwait(0, 0, isem0)
        _gissue(0)
        _ld(1, 1, isem1)

        def _pair(kk, _):
            for b in range(2):
                k = kk * 2 + b
                i_this, i_next = (isem0, isem1) if b == 0 else (isem1, isem0)
                _gwait(b)

                @pl.when(k < _TCH - 1)
                def _():
                    _ldwait(k + 1, 1 - b, i_next)
                    _gissue(1 - b)

                def _grp(g, _):
                    wv = ewv[b, pl.ds(g * 16, 16)]
                    e0 = g * 16
                    for l in range(16):
                        w = jnp.take_along_axis(
                            wv, jnp.full((16,), l, jnp.int32), axis=0)
                        for j in range(_D // 16):
                            sl = pl.ds(j * 16, 16)
                            rows[b, e0 + l, sl] = rows[b, e0 + l, sl] * w
                    return 0

                lax.fori_loop(0, _ACH // 16, _grp, 0)
                pltpu.sync_copy(rows.at[b], acc.at[dsti.at[b]], add=True)

                @pl.when(k < _TCH - 2)
                def _():
                    _ld(k + 2, b, i_this)
            return 0

        lax.fori_loop(0, _TCH // 2, _pair, 0)

    @pl.when(c == 0)
    def _():
        _run(u1, src1, dst1, ew1)

    @pl.when(c == 1)
    def _():
        _run(u2, src2, dst2, ew2)

    plsc.subcore_barrier()

    @pl.when(c == 0)
    def _():
        pltpu.sync_copy(acc.at[pl.ds(s * _PADROWS_PT, _PADROWS_PT)],
                        v1.at[pl.ds(s * _PADROWS_PT, _PADROWS_PT)])

    @pl.when(c == 1)
    def _():
        pltpu.sync_copy(acc.at[pl.ds(s * _PADROWS_PT, _PADROWS_PT)],
                        v2.at[pl.ds(s * _PADROWS_PT, _PADROWS_PT)])


@functools.cache
def _agg_call():
    return pl.kernel(
        _agg_body,
        out_type=[jax.ShapeDtypeStruct((_NP, _D), jnp.float32),
                  jax.ShapeDtypeStruct((_NP, _D), jnp.float32)],
        mesh=_sc_mesh(),
        scratch_types=[
            pltpu.VMEM_SHARED((_NP, _D), jnp.float32),
            pltpu.VMEM((2, _ACH), jnp.int32),
            pltpu.VMEM((2, _ACH), jnp.int32),
            pltpu.VMEM((2, _ACH), jnp.float32),
            pltpu.VMEM((2, _ACH, _D), jnp.float32),
        ] + [pltpu.SemaphoreType.DMA] * 10,
    )


def _pad1d(x):
    pad = jnp.zeros((_EPAD - _E,), x.dtype)
    return jnp.concatenate([x, pad])


_BLK = 1000


def _lin_body(deg_ref, fm_ref, w_ref, u_ref, dinv_ref):
    dinv = lax.rsqrt(deg_ref[...])
    u_ref[...] = dinv * jnp.dot(fm_ref[...], w_ref[...],
                                preferred_element_type=jnp.float32)
    dinv_ref[...] = dinv


def _lin(deg2d, fm, W):
    return pl.pallas_call(
        _lin_body,
        grid=(_N // _BLK,),
        in_specs=[
            pl.BlockSpec((_BLK, 1), lambda i: (i, 0)),
            pl.BlockSpec((_BLK, _D), lambda i: (i, 0)),
            pl.BlockSpec((_D, _D), lambda i: (0, 0)),
        ],
        out_specs=[
            pl.BlockSpec((_BLK, _D), lambda i: (i, 0)),
            pl.BlockSpec((_BLK, 1), lambda i: (i, 0)),
        ],
        out_shape=[jax.ShapeDtypeStruct((_N, _D), jnp.float32),
                   jax.ShapeDtypeStruct((_N, 1), jnp.float32)],
    )(deg2d, fm, W)


def _layer_body(dinv_ref, v_ref, u_ref, b_ref, w2_ref, x_ref, u2_ref):
    dinv = dinv_ref[...]
    x = dinv * (v_ref[...] + u_ref[...]) + b_ref[...]
    x_ref[...] = x
    u2_ref[...] = dinv * jnp.dot(x, w2_ref[...],
                                 preferred_element_type=jnp.float32)


def _layer(dinv, v, u, b, W2):
    return pl.pallas_call(
        _layer_body,
        grid=(_N // _BLK,),
        in_specs=[
            pl.BlockSpec((_BLK, 1), lambda i: (i, 0)),
            pl.BlockSpec((_BLK, _D), lambda i: (i, 0)),
            pl.BlockSpec((_BLK, _D), lambda i: (i, 0)),
            pl.BlockSpec((1, _D), lambda i: (0, 0)),
            pl.BlockSpec((_D, _D), lambda i: (0, 0)),
        ],
        out_specs=[
            pl.BlockSpec((_BLK, _D), lambda i: (i, 0)),
            pl.BlockSpec((_BLK, _D), lambda i: (i, 0)),
        ],
        out_shape=[jax.ShapeDtypeStruct((_N, _D), jnp.float32),
                   jax.ShapeDtypeStruct((_N, _D), jnp.float32)],
    )(dinv, v, u, b, W2)


_G = 64


def _final_body(x11_ref, x12_ref, v21_ref, u21_ref, dinv1_ref,
                v22_ref, u22_ref, dinv2_ref, b2_ref, batch_ref,
                m1_ref, bm1_ref, m2_ref, bm2_ref, m3_ref, bm3_ref,
                m4_ref, bm4_ref, out_ref, sums, cnt):
    i = pl.program_id(0)

    @pl.when(i == 0)
    def _():
        sums[...] = jnp.zeros_like(sums)
        cnt[...] = jnp.zeros_like(cnt)

    x21 = dinv1_ref[...] * (v21_ref[...] + u21_ref[...]) + b2_ref[...]
    x22 = dinv2_ref[...] * (v22_ref[...] + u22_ref[...]) + b2_ref[...]
    x = (x12_ref[...] - x11_ref[...]) * (x22 - x21)

    gids = lax.broadcasted_iota(jnp.int32, (1, _G), 1)
    oh = (batch_ref[...] == gids).astype(jnp.float32)
    dn = (((0,), (0,)), ((), ()))
    sums[...] += lax.dot_general(oh, x, dn, preferred_element_type=jnp.float32)
    cnt[...] += lax.dot_general(oh, jnp.ones_like(x), dn,
                                preferred_element_type=jnp.float32)

    @pl.when(i == (_N // _BLK) - 1)
    def _():
        pooled = sums[...] / jnp.maximum(cnt[...], 1.0)
        h = jnp.dot(pooled, m1_ref[...], preferred_element_type=jnp.float32) + bm1_ref[...]
        h = jnp.dot(h, m2_ref[...], preferred_element_type=jnp.float32) + bm2_ref[...]
        h = jnp.dot(h, m3_ref[...], preferred_element_type=jnp.float32) + bm3_ref[...]
        h = jnp.dot(h, m4_ref[...], preferred_element_type=jnp.float32) + bm4_ref[...]
        out_ref[...] = h


def _final(x11, x12, v21, u21, dinv1, v22, u22, dinv2, b2, batch2d,
           M1, bm1, M2, bm2, M3, bm3, M4, bm4):
    row = lambda i: (i, 0)
    full = lambda i: (0, 0)
    return pl.pallas_call(
        _final_body,
        grid=(_N // _BLK,),
        in_specs=[
            pl.BlockSpec((_BLK, _D), row),
            pl.BlockSpec((_BLK, _D), row),
            pl.BlockSpec((_BLK, _D), row),
            pl.BlockSpec((_BLK, _D), row),
            pl.BlockSpec((_BLK, 1), row),
            pl.BlockSpec((_BLK, _D), row),
            pl.BlockSpec((_BLK, _D), row),
            pl.BlockSpec((_BLK, 1), row),
            pl.BlockSpec((1, _D), full),
            pl.BlockSpec((_BLK, 1), row),
            pl.BlockSpec((_D, _D), full),
            pl.BlockSpec((1, _D), full),
            pl.BlockSpec((_D, _D // 2), full),
            pl.BlockSpec((1, _D // 2), full),
            pl.BlockSpec((_D // 2, _D // 4), full),
            pl.BlockSpec((1, _D // 4), full),
            pl.BlockSpec((_D // 4, 1), full),
            pl.BlockSpec((1, 1), full),
        ],
        out_specs=pl.BlockSpec((_G, 1), full),
        out_shape=jax.ShapeDtypeStruct((_G, 1), jnp.float32),
        scratch_shapes=[
            pltpu.VMEM((_G, _D), jnp.float32),
            pltpu.VMEM((_G, _D), jnp.float32),
        ],
    )(x11, x12, v21, u21, dinv1, v22, u22, dinv2, b2, batch2d,
      M1, bm1, M2, bm2, M3, bm3, M4, bm4)


def kernel(edge_index1, edge_weight1, edge_index2, edge_weight2, fm0, fm1,
           batch_tensor, W1, b1, W2, b2, M1, bm1, M2, bm2, M3, bm3, M4, bm4):
    src1, dst1 = edge_index1[0], edge_index1[1]
    src2, dst2 = edge_index2[0], edge_index2[1]

    sp1, dp1, wp1 = _pad1d(src1), _pad1d(dst1), _pad1d(edge_weight1)
    sp2, dp2, wp2 = _pad1d(src2), _pad1d(dst2), _pad1d(edge_weight2)

    deg1p, deg2p = _deg_call()(dp1, wp1, dp2, wp2)
    deg1, deg2 = deg1p[:_N], deg2p[:_N]

    u11, dinv1 = _lin(deg1.reshape(_N, 1), fm0, W1)
    u12, dinv2 = _lin(deg2.reshape(_N, 1), fm1, W1)

    v11p, v12p = _agg_call()(u11, sp1, dp1, wp1, u12, sp2, dp2, wp2)
    v11, v12 = v11p[:_N], v12p[:_N]

    b1r = b1.reshape(1, _D)
    x11, u21 = _layer(dinv1, v11, u11, b1r, W2)
    x12, u22 = _layer(dinv2, v12, u12, b1r, W2)

    v21p, v22p = _agg_call()(u21, sp1, dp1, wp1, u22, sp2, dp2, wp2)
    v21, v22 = v21p[:_N], v22p[:_N]

    return _final(x11, x12, v21, u21, dinv1, v22, u22, dinv2,
                  b2.reshape(1, _D), batch_tensor.reshape(_N, 1),
                  M1, bm1.reshape(1, _D), M2, bm2.reshape(1, _D // 2),
                  M3, bm3.reshape(1, _D // 4), M4, bm4.reshape(1, 1))


# speedup vs baseline: 1.1780x; 1.0452x over previous
import functools

import jax
import jax.numpy as jnp
from jax import lax
from jax.experimental import pallas as pl
from jax.experimental.pallas import tpu as pltpu
from jax.experimental.pallas import tpu_sc as plsc

_NC = 2
_NS = 16
_N = 10000
_E = 320000
_D = 128
_NP = 10240

_ROWS_PT = _N // _NS
_PADROWS_PT = _NP // _NS

@functools.cache
def _sc_mesh():
    return plsc.VectorSubcoreMesh(
        core_axis_name="c", subcore_axis_name="s",
        num_cores=_NC, num_subcores=_NS,
    )


_DCH = 128
_DTC = 327680 // (16 * _DCH)


def _deg_body(dst1, ew1, dst2, ew2, deg1, deg2, acc, idxr, valr,
              isem0, isem1):
    c = lax.axis_index("c")
    s = lax.axis_index("s")

    def _fill(i, _):
        valr[0, pl.ds(i * 16, 16)] = jnp.full((16,), 1.0, jnp.float32)
        return 0

    lax.fori_loop(0, _DCH // 16, _fill, 0)

    def _init(i, _):
        pltpu.sync_copy(valr.at[0],
                        acc.at[pl.ds(s * _PADROWS_PT + i * _DCH, _DCH)])
        return 0

    lax.fori_loop(0, _PADROWS_PT // _DCH, _init, 0)
    plsc.subcore_barrier()

    def _run(dst_hbm, ew_hbm):
        ebase = s * _DTC * _DCH

        def _ld(k, slot, sem):
            off = ebase + k * _DCH
            pltpu.async_copy(dst_hbm.at[pl.ds(off, _DCH)], idxr.at[slot], sem)
            pltpu.async_copy(ew_hbm.at[pl.ds(off, _DCH)], valr.at[slot], sem)

        def _ldwait(k, slot, sem):
            off = ebase + k * _DCH
            pltpu.make_async_copy(
                dst_hbm.at[pl.ds(off, _DCH)], idxr.at[slot], sem).wait()
            pltpu.make_async_copy(
                ew_hbm.at[pl.ds(off, _DCH)], valr.at[slot], sem).wait()

        _ld(0, 0, isem0)
        _ld(1, 1, isem1)

        def _pair(kk, _):
            for b in range(2):
                k = kk * 2 + b
                i_this = isem0 if b == 0 else isem1
                _ldwait(k, b, i_this)
                pltpu.sync_copy(valr.at[b], acc.at[idxr.at[b]], add=True)

                @pl.when(k < _DTC - 2)
                def _():
                    _ld(k + 2, b, i_this)
            return 0

        lax.fori_loop(0, _DTC // 2, _pair, 0)

    @pl.when(c == 0)
    def _():
        _run(dst1, ew1)

    @pl.when(c == 1)
    def _():
        _run(dst2, ew2)

    plsc.subcore_barrier()

    @pl.when(c == 0)
    def _():
        pltpu.sync_copy(acc.at[pl.ds(s * _PADROWS_PT, _PADROWS_PT)],
                        deg1.at[pl.ds(s * _PADROWS_PT, _PADROWS_PT)])

    @pl.when(c == 1)
    def _():
        pltpu.sync_copy(acc.at[pl.ds(s * _PADROWS_PT, _PADROWS_PT)],
                        deg2.at[pl.ds(s * _PADROWS_PT, _PADROWS_PT)])


@functools.cache
def _deg_call():
    return pl.kernel(
        _deg_body,
        out_type=[jax.ShapeDtypeStruct((_NP,), jnp.float32),
                  jax.ShapeDtypeStruct((_NP,), jnp.float32)],
        mesh=_sc_mesh(),
        scratch_types=[
            pltpu.VMEM_SHARED((_NP,), jnp.float32),
            pltpu.VMEM((2, _DCH), jnp.int32),
            pltpu.VMEM((2, _DCH), jnp.float32),
            pltpu.SemaphoreType.DMA,
            pltpu.SemaphoreType.DMA,
        ],
    )


_ACH = 128
_EPAD = 327680
_ER = _EPAD // _ACH
_TCH = _ER // _NS


_NSUB = 4
_SUBR = _ACH // _NSUB


def _agg_body(u1, src1, dst1, ew1, u2, src2, dst2, ew2, v1, v2,
              acc, srci, dsti, ewv, rows,
              gs0, gs1, gs2, gs3, gs4, gs5, gs6, gs7,
              is0, is1, is2, is3, ss0, ss1):
    gsems = ((gs0, gs1, gs2, gs3), (gs4, gs5, gs6, gs7))
    isems = (is0, is1, is2, is3)
    ssems = (ss0, ss1)
    c = lax.axis_index("c")
    s = lax.axis_index("s")

    def _zrow(r, _):
        for j in range(_D // 16):
            rows[0, r, pl.ds(j * 16, 16)] = jnp.zeros((16,), jnp.float32)
        return 0

    lax.fori_loop(0, _ACH, _zrow, 0)

    def _zinit(i, _):
        pltpu.sync_copy(rows.at[0],
                        acc.at[pl.ds(s * _PADROWS_PT + i * _ACH, _ACH)])
        return 0

    lax.fori_loop(0, _PADROWS_PT // _ACH, _zinit, 0)
    plsc.subcore_barrier()

    def _run(u_hbm, src_hbm, dst_hbm, ew_hbm):
        ebase = s * _TCH * _ACH

        def _ld(k, slot, sem):
            off = ebase + k * _ACH
            pltpu.async_copy(src_hbm.at[pl.ds(off, _ACH)], srci.at[slot], sem)
            pltpu.async_copy(dst_hbm.at[pl.ds(off, _ACH)], dsti.at[slot], sem)
            pltpu.async_copy(ew_hbm.at[pl.ds(off, _ACH)], ewv.at[slot], sem)

        def _ldwait(k, slot, sem):
            off = ebase + k * _ACH
            pltpu.make_async_copy(
                src_hbm.at[pl.ds(off, _ACH)], srci.at[slot], sem).wait()
            pltpu.make_async_copy(
                dst_hbm.at[pl.ds(off, _ACH)], dsti.at[slot], sem).wait()
            pltpu.make_async_copy(
                ew_hbm.at[pl.ds(off, _ACH)], ewv.at[slot], sem).wait()

        def _gissue(p, islot):
            for q in range(_NSUB):
                sl = pl.ds(q * _SUBR, _SUBR)
                pltpu.async_copy(u_hbm.at[srci.at[islot, sl]],
                                 rows.at[p, sl], gsems[p][q])

        def _gwait(p, islot):
            for q in range(_NSUB):
                sl = pl.ds(q * _SUBR, _SUBR)
                pltpu.make_async_copy(u_hbm.at[srci.at[islot, sl]],
                                      rows.at[p, sl],
                                      gsems[p][q]).wait()

        _ld(0, 0, isems[0])
        _ldwait(0, 0, isems[0])
        _gissue(0, 0)
        _ld(1, 1, isems[1])

        def _quad(kk, _):
            for b in range(4):
                k = kk * 4 + b
                p = b % 2
                _gwait(p, b)

                @pl.when(k < _TCH - 1)
                def _():
                    @pl.when(k >= 1)
                    def _():
                        pltpu.make_async_copy(
                            rows.at[1 - p], acc.at[dsti.at[(b - 1) % 4]],
                            ssems[1 - p]).wait()
                    _ldwait(k + 1, (b + 1) % 4, isems[(b + 1) % 4])
                    _gissue(1 - p, (b + 1) % 4)

                def _grp(g, _):
                    wv = ewv[b % 4, pl.ds(g * 16, 16)]
                    e0 = g * 16
                    for l in range(16):
                        w = jnp.take_along_axis(
                            wv, jnp.full((16,), l, jnp.int32), axis=0)
                        for j in range(_D // 16):
                            sl = pl.ds(j * 16, 16)
                            rows[p, e0 + l, sl] = rows[p, e0 + l, sl] * w
                    return 0

                lax.fori_loop(0, _ACH // 16, _grp, 0)
                pltpu.async_copy(rows.at[p], acc.at[dsti.at[b]],
                                 ssems[p], add=True)

                @pl.when(k < _TCH - 2)
                def _():
                    _ld(k + 2, (b + 2) % 4, isems[(b + 2) % 4])
            return 0

        lax.fori_loop(0, _TCH // 4, _quad, 0)
        pltpu.make_async_copy(rows.at[0], acc.at[dsti.at[(_TCH - 2) % 4]],
                              ssems[0]).wait()
        pltpu.make_async_copy(rows.at[1], acc.at[dsti.at[(_TCH - 1) % 4]],
                              ssems[1]).wait()

    @pl.when(c == 0)
    def _():
        _run(u1, src1, dst1, ew1)

    @pl.when(c == 1)
    def _():
        _run(u2, src2, dst2, ew2)

    plsc.subcore_barrier()

    @pl.when(c == 0)
    def _():
        pltpu.sync_copy(acc.at[pl.ds(s * _PADROWS_PT, _PADROWS_PT)],
                        v1.at[pl.ds(s * _PADROWS_PT, _PADROWS_PT)])

    @pl.when(c == 1)
    def _():
        pltpu.sync_copy(acc.at[pl.ds(s * _PADROWS_PT, _PADROWS_PT)],
                        v2.at[pl.ds(s * _PADROWS_PT, _PADROWS_PT)])


@functools.cache
def _agg_call():
    return pl.kernel(
        _agg_body,
        out_type=[jax.ShapeDtypeStruct((_NP, _D), jnp.float32),
                  jax.ShapeDtypeStruct((_NP, _D), jnp.float32)],
        mesh=_sc_mesh(),
        scratch_types=[
            pltpu.VMEM_SHARED((_NP, _D), jnp.float32),
            pltpu.VMEM((4, _ACH), jnp.int32),
            pltpu.VMEM((4, _ACH), jnp.int32),
            pltpu.VMEM((4, _ACH), jnp.float32),
            pltpu.VMEM((2, _ACH, _D), jnp.float32),
        ] + [pltpu.SemaphoreType.DMA] * 14,
    )


def _pad1d(x):
    pad = jnp.zeros((_EPAD - _E,), x.dtype)
    return jnp.concatenate([x, pad])


_BLK = 1000


def _lin_body(deg_ref, fm_ref, w_ref, u_ref, dinv_ref):
    dinv = lax.rsqrt(deg_ref[...])
    u_ref[...] = dinv * jnp.dot(fm_ref[...], w_ref[...],
                                preferred_element_type=jnp.float32)
    dinv_ref[...] = dinv


def _lin(deg2d, fm, W):
    return pl.pallas_call(
        _lin_body,
        grid=(_N // _BLK,),
        in_specs=[
            pl.BlockSpec((_BLK, 1), lambda i: (i, 0)),
            pl.BlockSpec((_BLK, _D), lambda i: (i, 0)),
            pl.BlockSpec((_D, _D), lambda i: (0, 0)),
        ],
        out_specs=[
            pl.BlockSpec((_BLK, _D), lambda i: (i, 0)),
            pl.BlockSpec((_BLK, 1), lambda i: (i, 0)),
        ],
        out_shape=[jax.ShapeDtypeStruct((_N, _D), jnp.float32),
                   jax.ShapeDtypeStruct((_N, 1), jnp.float32)],
    )(deg2d, fm, W)


def _layer_body(dinv_ref, v_ref, u_ref, b_ref, w2_ref, x_ref, u2_ref):
    dinv = dinv_ref[...]
    x = dinv * (v_ref[...] + u_ref[...]) + b_ref[...]
    x_ref[...] = x
    u2_ref[...] = dinv * jnp.dot(x, w2_ref[...],
                                 preferred_element_type=jnp.float32)


def _layer(dinv, v, u, b, W2):
    return pl.pallas_call(
        _layer_body,
        grid=(_N // _BLK,),
        in_specs=[
            pl.BlockSpec((_BLK, 1), lambda i: (i, 0)),
            pl.BlockSpec((_BLK, _D), lambda i: (i, 0)),
            pl.BlockSpec((_BLK, _D), lambda i: (i, 0)),
            pl.BlockSpec((1, _D), lambda i: (0, 0)),
            pl.BlockSpec((_D, _D), lambda i: (0, 0)),
        ],
        out_specs=[
            pl.BlockSpec((_BLK, _D), lambda i: (i, 0)),
            pl.BlockSpec((_BLK, _D), lambda i: (i, 0)),
        ],
        out_shape=[jax.ShapeDtypeStruct((_N, _D), jnp.float32),
                   jax.ShapeDtypeStruct((_N, _D), jnp.float32)],
    )(dinv, v, u, b, W2)


_G = 64


def _final_body(x11_ref, x12_ref, v21_ref, u21_ref, dinv1_ref,
                v22_ref, u22_ref, dinv2_ref, b2_ref, batch_ref,
                m1_ref, bm1_ref, m2_ref, bm2_ref, m3_ref, bm3_ref,
                m4_ref, bm4_ref, out_ref, sums, cnt):
    i = pl.program_id(0)

    @pl.when(i == 0)
    def _():
        sums[...] = jnp.zeros_like(sums)
        cnt[...] = jnp.zeros_like(cnt)

    x21 = dinv1_ref[...] * (v21_ref[...] + u21_ref[...]) + b2_ref[...]
    x22 = dinv2_ref[...] * (v22_ref[...] + u22_ref[...]) + b2_ref[...]
    x = (x12_ref[...] - x11_ref[...]) * (x22 - x21)

    gids = lax.broadcasted_iota(jnp.int32, (1, _G), 1)
    oh = (batch_ref[...] == gids).astype(jnp.float32)
    dn = (((0,), (0,)), ((), ()))
    sums[...] += lax.dot_general(oh, x, dn, preferred_element_type=jnp.float32)
    cnt[...] += lax.dot_general(oh, jnp.ones_like(x), dn,
                                preferred_element_type=jnp.float32)

    @pl.when(i == (_N // _BLK) - 1)
    def _():
        pooled = sums[...] / jnp.maximum(cnt[...], 1.0)
        h = jnp.dot(pooled, m1_ref[...], preferred_element_type=jnp.float32) + bm1_ref[...]
        h = jnp.dot(h, m2_ref[...], preferred_element_type=jnp.float32) + bm2_ref[...]
        h = jnp.dot(h, m3_ref[...], preferred_element_type=jnp.float32) + bm3_ref[...]
        h = jnp.dot(h, m4_ref[...], preferred_element_type=jnp.float32) + bm4_ref[...]
        out_ref[...] = h


def _final(x11, x12, v21, u21, dinv1, v22, u22, dinv2, b2, batch2d,
           M1, bm1, M2, bm2, M3, bm3, M4, bm4):
    row = lambda i: (i, 0)
    full = lambda i: (0, 0)
    return pl.pallas_call(
        _final_body,
        grid=(_N // _BLK,),
        in_specs=[
            pl.BlockSpec((_BLK, _D), row),
            pl.BlockSpec((_BLK, _D), row),
            pl.BlockSpec((_BLK, _D), row),
            pl.BlockSpec((_BLK, _D), row),
            pl.BlockSpec((_BLK, 1), row),
            pl.BlockSpec((_BLK, _D), row),
            pl.BlockSpec((_BLK, _D), row),
            pl.BlockSpec((_BLK, 1), row),
            pl.BlockSpec((1, _D), full),
            pl.BlockSpec((_BLK, 1), row),
            pl.BlockSpec((_D, _D), full),
            pl.BlockSpec((1, _D), full),
            pl.BlockSpec((_D, _D // 2), full),
            pl.BlockSpec((1, _D // 2), full),
            pl.BlockSpec((_D // 2, _D // 4), full),
            pl.BlockSpec((1, _D // 4), full),
            pl.BlockSpec((_D // 4, 1), full),
            pl.BlockSpec((1, 1), full),
        ],
        out_specs=pl.BlockSpec((_G, 1), full),
        out_shape=jax.ShapeDtypeStruct((_G, 1), jnp.float32),
        scratch_shapes=[
            pltpu.VMEM((_G, _D), jnp.float32),
            pltpu.VMEM((_G, _D), jnp.float32),
        ],
    )(x11, x12, v21, u21, dinv1, v22, u22, dinv2, b2, batch2d,
      M1, bm1, M2, bm2, M3, bm3, M4, bm4)


def kernel(edge_index1, edge_weight1, edge_index2, edge_weight2, fm0, fm1,
           batch_tensor, W1, b1, W2, b2, M1, bm1, M2, bm2, M3, bm3, M4, bm4):
    src1, dst1 = edge_index1[0], edge_index1[1]
    src2, dst2 = edge_index2[0], edge_index2[1]

    sp1, dp1, wp1 = _pad1d(src1), _pad1d(dst1), _pad1d(edge_weight1)
    sp2, dp2, wp2 = _pad1d(src2), _pad1d(dst2), _pad1d(edge_weight2)

    deg1p, deg2p = _deg_call()(dp1, wp1, dp2, wp2)
    deg1, deg2 = deg1p[:_N], deg2p[:_N]

    u11, dinv1 = _lin(deg1.reshape(_N, 1), fm0, W1)
    u12, dinv2 = _lin(deg2.reshape(_N, 1), fm1, W1)

    v11p, v12p = _agg_call()(u11, sp1, dp1, wp1, u12, sp2, dp2, wp2)
    v11, v12 = v11p[:_N], v12p[:_N]

    b1r = b1.reshape(1, _D)
    x11, u21 = _layer(dinv1, v11, u11, b1r, W2)
    x12, u22 = _layer(dinv2, v12, u12, b1r, W2)

    v21p, v22p = _agg_call()(u21, sp1, dp1, wp1, u22, sp2, dp2, wp2)
    v21, v22 = v21p[:_N], v22p[:_N]

    return _final(x11, x12, v21, u21, dinv1, v22, u22, dinv2,
                  b2.reshape(1, _D), batch_tensor.reshape(_N, 1),
                  M1, bm1.reshape(1, _D), M2, bm2.reshape(1, _D // 2),
                  M3, bm3.reshape(1, _D // 4), M4, bm4.reshape(1, 1))
